# baseline (speedup 1.0000x reference)
import jax
import jax.numpy as jnp
from jax import lax
from jax.experimental import pallas as pl
from jax.experimental.pallas import tpu as pltpu
from jax.experimental.pallas import tpu_sc as plsc

N = 10000
D = 128
N_CLS = 64
E = 320000

NC = 2
NS = 16
NW = NC * NS
L = 16

CHUNK = 128
NCH = 80
EP = NW * CHUNK * NCH
XCH = 2
RPW = 640
N_PAD = NS * RPW

_MESH = plsc.VectorSubcoreMesh(core_axis_name="c", subcore_axis_name="s")


def _sc_agg_body(p_hbm, srcc, dstc, agg_out,
                 idx_s, idx_d, rows, accum,
                 sis0, sis1, sid0, sid1, sg0, sg1):
    c = lax.axis_index("c")
    s = lax.axis_index("s")
    w = c * NS + s
    base = w * NCH
    zeros16 = jnp.zeros((L,), jnp.float32)
    zbuf = rows.at[0]

    @pl.loop(0, CHUNK)
    def _(i):
        for j in range(D // L):
            zbuf[i, pl.ds(j * L, L)] = zeros16

    @pl.loop(0, RPW // CHUNK)
    def _(k):
        pltpu.sync_copy(zbuf, accum.at[pl.ds(s * RPW + k * CHUNK, CHUNK)])

    plsc.subcore_barrier()

    pltpu.sync_copy(srcc.at[base], idx_s.at[0])
    pltpu.sync_copy(dstc.at[base], idx_d.at[0])
    pltpu.async_copy(p_hbm.at[idx_s.at[0]], rows.at[0], sg0)
    pltpu.async_copy(srcc.at[base + 1], idx_s.at[1], sis1)
    pltpu.async_copy(dstc.at[base + 1], idx_d.at[1], sid1)

    @pl.loop(0, NCH, step=2)
    def _(g):
        pltpu.make_async_copy(srcc.at[base + g + 1], idx_s.at[1], sis1).wait()
        pltpu.async_copy(p_hbm.at[idx_s.at[1]], rows.at[1], sg1)
        pltpu.make_async_copy(p_hbm.at[idx_s.at[0]], rows.at[0], sg0).wait()
        pltpu.make_async_copy(dstc.at[base + g + 1], idx_d.at[1], sid1).wait()
        pltpu.sync_copy(rows.at[0], accum.at[idx_d.at[0]], add=True)
        pltpu.async_copy(srcc.at[base + g + 2], idx_s.at[0], sis0)
        pltpu.async_copy(dstc.at[base + g + 2], idx_d.at[0], sid0)

        pltpu.make_async_copy(srcc.at[base + g + 2], idx_s.at[0], sis0).wait()
        pltpu.async_copy(p_hbm.at[idx_s.at[0]], rows.at[0], sg0)
        pltpu.make_async_copy(p_hbm.at[idx_s.at[1]], rows.at[1], sg1).wait()
        pltpu.make_async_copy(dstc.at[base + g + 2], idx_d.at[0], sid0).wait()
        pltpu.sync_copy(rows.at[1], accum.at[idx_d.at[1]], add=True)
        pltpu.async_copy(srcc.at[base + g + 3], idx_s.at[1], sis1)
        pltpu.async_copy(dstc.at[base + g + 3], idx_d.at[1], sid1)

    pltpu.make_async_copy(p_hbm.at[idx_s.at[0]], rows.at[0], sg0).wait()
    pltpu.make_async_copy(srcc.at[base + NCH + 1], idx_s.at[1], sis1).wait()
    pltpu.make_async_copy(dstc.at[base + NCH + 1], idx_d.at[1], sid1).wait()

    plsc.subcore_barrier()

    @pl.loop(0, RPW // CHUNK)
    def _(k):
        r0 = s * RPW + k * CHUNK
        pltpu.sync_copy(accum.at[pl.ds(r0, CHUNK)], rows.at[0])
        pltpu.sync_copy(rows.at[0], agg_out.at[pl.ds(c * N_PAD + r0, CHUNK)])


def _sc_deg_body(dstc, deg_out, idx_d, ones, deg_sh):
    c = lax.axis_index("c")
    s = lax.axis_index("s")
    w = c * NS + s
    zeros16 = jnp.zeros((L,), jnp.float32)

    @pl.loop(0, CHUNK)
    def _(i):
        for j in range(D // L):
            ones[i, pl.ds(j * L, L)] = zeros16

    @pl.loop(0, RPW // CHUNK)
    def _(k):
        pltpu.sync_copy(ones, deg_sh.at[pl.ds(s * RPW + k * CHUNK, CHUNK)])

    @pl.loop(0, CHUNK)
    def _(i):
        for j in range(D // L):
            ones[i, pl.ds(j * L, L)] = zeros16 + 1.0

    plsc.subcore_barrier()

    @pl.loop(0, NCH)
    def _(g):
        pltpu.sync_copy(dstc.at[w * NCH + g], idx_d.at[0])
        pltpu.sync_copy(ones, deg_sh.at[idx_d.at[0]], add=True)

    plsc.subcore_barrier()

    @pl.loop(0, RPW // CHUNK)
    def _(k):
        r0 = s * RPW + k * CHUNK
        pltpu.sync_copy(deg_sh.at[pl.ds(r0, CHUNK)], ones)
        pltpu.sync_copy(ones, deg_out.at[pl.ds(c * N_PAD + r0, CHUNK)])


_sc_agg = pl.kernel(
    _sc_agg_body,
    out_type=jax.ShapeDtypeStruct((NC * N_PAD, D), jnp.float32),
    mesh=_MESH,
    scratch_types=[
        pltpu.VMEM((2, CHUNK), jnp.int32),
        pltpu.VMEM((2, CHUNK), jnp.int32),
        pltpu.VMEM((2, CHUNK, D), jnp.float32),
        pltpu.VMEM_SHARED((N_PAD, D), jnp.float32),
        pltpu.SemaphoreType.DMA,
        pltpu.SemaphoreType.DMA,
        pltpu.SemaphoreType.DMA,
        pltpu.SemaphoreType.DMA,
        pltpu.SemaphoreType.DMA,
        pltpu.SemaphoreType.DMA,
    ],
)

_sc_deg = pl.kernel(
    _sc_deg_body,
    out_type=jax.ShapeDtypeStruct((NC * N_PAD, D), jnp.float32),
    mesh=_MESH,
    scratch_types=[
        pltpu.VMEM((1, CHUNK), jnp.int32),
        pltpu.VMEM((CHUNK, D), jnp.float32),
        pltpu.VMEM_SHARED((N_PAD, D), jnp.float32),
    ],
)


def _proj_kernel(x_ref, wn_ref, ws_ref, b_ref, p_ref, s_ref):
    x = x_ref[...]
    p_ref[...] = jnp.dot(x, wn_ref[...], preferred_element_type=jnp.float32)
    s_ref[...] = (
        jnp.dot(x, ws_ref[...], preferred_element_type=jnp.float32) + b_ref[...]
    )


def _mid_kernel(s_ref, a_ref, d_ref, wn_ref, ws_ref, b_ref, p_ref, s2_ref):
    deg = jnp.maximum(d_ref[0, :, 0:1] + d_ref[1, :, 0:1], 1.0)
    agg = (a_ref[0] + a_ref[1]) / deg
    h = jnp.maximum(s_ref[...] + agg, 0.0)
    p_ref[...] = jnp.dot(h, wn_ref[...], preferred_element_type=jnp.float32)
    s2_ref[...] = (
        jnp.dot(h, ws_ref[...], preferred_element_type=jnp.float32) + b_ref[...]
    )


def _head_kernel(s_ref, a_ref, d_ref, wfc_ref, b_ref, o_ref):
    deg = jnp.maximum(d_ref[0, :, 0:1] + d_ref[1, :, 0:1], 1.0)
    agg = (a_ref[0] + a_ref[1]) / deg
    h = jnp.maximum(s_ref[...] + agg, 0.0)
    o_ref[...] = (
        jnp.dot(h, wfc_ref[...], preferred_element_type=jnp.float32) + b_ref[...]
    )


_BR = 1000


def _row_spec(d):
    return pl.BlockSpec((_BR, d), lambda i: (i, 0))


def _part_spec(d):
    return pl.BlockSpec((2, _BR, d), lambda i: (0, i, 0))


def _full_spec(a, b):
    return pl.BlockSpec((a, b), lambda i: (0, 0))


def kernel(x, edge_index, W_self1, W_neigh1, b1, W_self2, W_neigh2, b2, W_fc, b_fc):
    src = edge_index[0].astype(jnp.int32)
    dst = edge_index[1].astype(jnp.int32)
    pad = EP - E
    xtra = XCH * CHUNK
    src_p = jnp.concatenate(
        [src, jnp.zeros((pad + xtra,), jnp.int32)]).reshape(NW * NCH + XCH, CHUNK)
    dst_p = jnp.concatenate(
        [dst, jnp.full((pad,), N, jnp.int32),
         jnp.zeros((xtra,), jnp.int32)]).reshape(NW * NCH + XCH, CHUNK)
    b1r = b1.reshape(1, D)
    b2r = b2.reshape(1, D)
    bfr = b_fc.reshape(1, N_CLS)

    proj = pl.pallas_call(
        _proj_kernel,
        grid=(N // _BR,),
        in_specs=[_row_spec(D), _full_spec(D, D), _full_spec(D, D),
                  _full_spec(1, D)],
        out_specs=[_row_spec(D), _row_spec(D)],
        out_shape=[jax.ShapeDtypeStruct((N, D), jnp.float32)] * 2,
    )
    p1, s1 = proj(x, W_neigh1, W_self1, b1r)

    deg3 = _sc_deg(dst_p).reshape(NC, N_PAD, D)
    agg1 = _sc_agg(p1, src_p, dst_p).reshape(NC, N_PAD, D)

    mid = pl.pallas_call(
        _mid_kernel,
        grid=(N // _BR,),
        in_specs=[_row_spec(D), _part_spec(D), _part_spec(D),
                  _full_spec(D, D), _full_spec(D, D), _full_spec(1, D)],
        out_specs=[_row_spec(D), _row_spec(D)],
        out_shape=[jax.ShapeDtypeStruct((N, D), jnp.float32)] * 2,
    )
    p2, s2 = mid(s1, agg1, deg3, W_neigh2, W_self2, b2r)

    agg2 = _sc_agg(p2, src_p, dst_p).reshape(NC, N_PAD, D)

    head = pl.pallas_call(
        _head_kernel,
        grid=(N // _BR,),
        in_specs=[_row_spec(D), _part_spec(D), _part_spec(D),
                  _full_spec(D, N_CLS), _full_spec(1, N_CLS)],
        out_specs=_row_spec(N_CLS),
        out_shape=jax.ShapeDtypeStruct((N, N_CLS), jnp.float32),
    )
    return head(s2, agg2, deg3, W_fc, bfr)


# speedup vs baseline: 2.4146x; 2.4146x over previous
import jax
import jax.numpy as jnp
from jax import lax
from jax.experimental import pallas as pl
from jax.experimental.pallas import tpu as pltpu
from jax.experimental.pallas import tpu_sc as plsc

N = 10000
D = 128
N_CLS = 64
E = 320000

NC = 2
NS = 16
NW = NC * NS
L = 16

CHUNK = 128
NCH = 80
EP = NW * CHUNK * NCH
XCH = 2
RPW = 640
N_PAD = NS * RPW

_MESH = plsc.VectorSubcoreMesh(core_axis_name="c", subcore_axis_name="s")


def _sc_agg_body(p_hbm, srcc, dstc, agg_out,
                 idx_s, idx_d, rows, accum,
                 sis0, sis1, sid0, sid1, sg0, sg1):
    c = lax.axis_index("c")
    s = lax.axis_index("s")
    w = c * NS + s
    base = w * NCH
    zeros16 = jnp.zeros((L,), jnp.float32)
    zbuf = rows.at[0]

    @pl.loop(0, CHUNK)
    def _(i):
        for j in range(D // L):
            zbuf[i, pl.ds(j * L, L)] = zeros16

    @pl.loop(0, RPW // CHUNK)
    def _(k):
        pltpu.sync_copy(zbuf, accum.at[pl.ds(s * RPW + k * CHUNK, CHUNK)])

    plsc.subcore_barrier()

    pltpu.sync_copy(srcc.at[base], idx_s.at[0])
    pltpu.sync_copy(dstc.at[base], idx_d.at[0])
    pltpu.async_copy(p_hbm.at[idx_s.at[0]], rows.at[0], sg0)
    pltpu.async_copy(srcc.at[base + 1], idx_s.at[1], sis1)
    pltpu.async_copy(dstc.at[base + 1], idx_d.at[1], sid1)

    @pl.loop(0, NCH, step=2)
    def _(g):
        pltpu.make_async_copy(srcc.at[base + g + 1], idx_s.at[1], sis1).wait()
        pltpu.async_copy(p_hbm.at[idx_s.at[1]], rows.at[1], sg1)
        pltpu.make_async_copy(p_hbm.at[idx_s.at[0]], rows.at[0], sg0).wait()
        pltpu.make_async_copy(dstc.at[base + g + 1], idx_d.at[1], sid1).wait()
        pltpu.sync_copy(rows.at[0], accum.at[idx_d.at[0]], add=True)
        pltpu.async_copy(srcc.at[base + g + 2], idx_s.at[0], sis0)
        pltpu.async_copy(dstc.at[base + g + 2], idx_d.at[0], sid0)

        pltpu.make_async_copy(srcc.at[base + g + 2], idx_s.at[0], sis0).wait()
        pltpu.async_copy(p_hbm.at[idx_s.at[0]], rows.at[0], sg0)
        pltpu.make_async_copy(p_hbm.at[idx_s.at[1]], rows.at[1], sg1).wait()
        pltpu.make_async_copy(dstc.at[base + g + 2], idx_d.at[0], sid0).wait()
        pltpu.sync_copy(rows.at[1], accum.at[idx_d.at[1]], add=True)
        pltpu.async_copy(srcc.at[base + g + 3], idx_s.at[1], sis1)
        pltpu.async_copy(dstc.at[base + g + 3], idx_d.at[1], sid1)

    pltpu.make_async_copy(p_hbm.at[idx_s.at[0]], rows.at[0], sg0).wait()
    pltpu.make_async_copy(srcc.at[base + NCH + 1], idx_s.at[1], sis1).wait()
    pltpu.make_async_copy(dstc.at[base + NCH + 1], idx_d.at[1], sid1).wait()

    plsc.subcore_barrier()

    @pl.loop(0, RPW // CHUNK)
    def _(k):
        r0 = s * RPW + k * CHUNK
        pltpu.sync_copy(accum.at[pl.ds(r0, CHUNK)], rows.at[0])
        pltpu.sync_copy(rows.at[0], agg_out.at[pl.ds(c * N_PAD + r0, CHUNK)])


def _sc_deg_body(dstc, deg_out, idx_d, ones, deg_sh):
    c = lax.axis_index("c")
    s = lax.axis_index("s")
    w = c * NS + s
    zeros16 = jnp.zeros((L,), jnp.float32)

    @pl.loop(0, CHUNK)
    def _(i):
        for j in range(D // L):
            ones[i, pl.ds(j * L, L)] = zeros16

    @pl.loop(0, RPW // CHUNK)
    def _(k):
        pltpu.sync_copy(ones, deg_sh.at[pl.ds(s * RPW + k * CHUNK, CHUNK)])

    @pl.loop(0, CHUNK)
    def _(i):
        for j in range(D // L):
            ones[i, pl.ds(j * L, L)] = zeros16 + 1.0

    plsc.subcore_barrier()

    @pl.loop(0, NCH)
    def _(g):
        pltpu.sync_copy(dstc.at[w * NCH + g], idx_d.at[0])
        pltpu.sync_copy(ones, deg_sh.at[idx_d.at[0]], add=True)

    plsc.subcore_barrier()

    @pl.loop(0, RPW // CHUNK)
    def _(k):
        r0 = s * RPW + k * CHUNK
        pltpu.sync_copy(deg_sh.at[pl.ds(r0, CHUNK)], ones)
        pltpu.sync_copy(ones, deg_out.at[pl.ds(c * N_PAD + r0, CHUNK)])


_sc_agg = pl.kernel(
    _sc_agg_body,
    out_type=jax.ShapeDtypeStruct((NC * N_PAD, D), jnp.float32),
    mesh=_MESH,
    scratch_types=[
        pltpu.VMEM((2, CHUNK), jnp.int32),
        pltpu.VMEM((2, CHUNK), jnp.int32),
        pltpu.VMEM((2, CHUNK, D), jnp.float32),
        pltpu.VMEM_SHARED((N_PAD, D), jnp.float32),
        pltpu.SemaphoreType.DMA,
        pltpu.SemaphoreType.DMA,
        pltpu.SemaphoreType.DMA,
        pltpu.SemaphoreType.DMA,
        pltpu.SemaphoreType.DMA,
        pltpu.SemaphoreType.DMA,
    ],
)

_sc_deg = pl.kernel(
    _sc_deg_body,
    out_type=jax.ShapeDtypeStruct((NC * N_PAD, D), jnp.float32),
    mesh=_MESH,
    scratch_types=[
        pltpu.VMEM((1, CHUNK), jnp.int32),
        pltpu.VMEM((CHUNK, D), jnp.float32),
        pltpu.VMEM_SHARED((N_PAD, D), jnp.float32),
    ],
)


def _proj_kernel(x_ref, wn_ref, ws_ref, b_ref, p_ref, s_ref):
    x = x_ref[...]
    p_ref[...] = jnp.dot(x, wn_ref[...], preferred_element_type=jnp.float32)
    s_ref[...] = (
        jnp.dot(x, ws_ref[...], preferred_element_type=jnp.float32) + b_ref[...]
    )


def _mid_kernel(s_ref, a_ref, d_ref, wn_ref, ws_ref, b_ref, p_ref, s2_ref):
    deg = jnp.maximum(d_ref[0, :, 0:1] + d_ref[1, :, 0:1], 1.0)
    agg = (a_ref[0] + a_ref[1]) / deg
    h = jnp.maximum(s_ref[...] + agg, 0.0)
    p_ref[...] = jnp.dot(h, wn_ref[...], preferred_element_type=jnp.float32)
    s2_ref[...] = (
        jnp.dot(h, ws_ref[...], preferred_element_type=jnp.float32) + b_ref[...]
    )


def _head_kernel(s_ref, a_ref, d_ref, wfc_ref, b_ref, o_ref):
    deg = jnp.maximum(d_ref[0, :, 0:1] + d_ref[1, :, 0:1], 1.0)
    agg = (a_ref[0] + a_ref[1]) / deg
    h = jnp.maximum(s_ref[...] + agg, 0.0)
    o_ref[...] = (
        jnp.dot(h, wfc_ref[...], preferred_element_type=jnp.float32) + b_ref[...]
    )


_BR = 1000


def _row_spec(d):
    return pl.BlockSpec((_BR, d), lambda i: (i, 0))


def _part_spec(d):
    return pl.BlockSpec((2, _BR, d), lambda i: (0, i, 0))


def _full_spec(a, b):
    return pl.BlockSpec((a, b), lambda i: (0, 0))


def kernel(x, edge_index, W_self1, W_neigh1, b1, W_self2, W_neigh2, b2, W_fc, b_fc):
    src = edge_index[0].astype(jnp.int32)
    dst = edge_index[1].astype(jnp.int32)
    pad = EP - E
    xtra = XCH * CHUNK
    pad_src = (jnp.arange(pad, dtype=jnp.int32) * 2003) % N
    pad_dst = N + (jnp.arange(pad, dtype=jnp.int32) % (N_PAD - N))
    src_p = jnp.concatenate(
        [src, pad_src, jnp.zeros((xtra,), jnp.int32)]).reshape(
            NW * NCH + XCH, CHUNK)
    dst_p = jnp.concatenate(
        [dst, pad_dst, jnp.zeros((xtra,), jnp.int32)]).reshape(
            NW * NCH + XCH, CHUNK)
    b1r = b1.reshape(1, D)
    b2r = b2.reshape(1, D)
    bfr = b_fc.reshape(1, N_CLS)

    proj = pl.pallas_call(
        _proj_kernel,
        grid=(N // _BR,),
        in_specs=[_row_spec(D), _full_spec(D, D), _full_spec(D, D),
                  _full_spec(1, D)],
        out_specs=[_row_spec(D), _row_spec(D)],
        out_shape=[jax.ShapeDtypeStruct((N, D), jnp.float32)] * 2,
    )
    p1, s1 = proj(x, W_neigh1, W_self1, b1r)

    deg3 = _sc_deg(dst_p).reshape(NC, N_PAD, D)
    agg1 = _sc_agg(p1, src_p, dst_p).reshape(NC, N_PAD, D)

    mid = pl.pallas_call(
        _mid_kernel,
        grid=(N // _BR,),
        in_specs=[_row_spec(D), _part_spec(D), _part_spec(D),
                  _full_spec(D, D), _full_spec(D, D), _full_spec(1, D)],
        out_specs=[_row_spec(D), _row_spec(D)],
        out_shape=[jax.ShapeDtypeStruct((N, D), jnp.float32)] * 2,
    )
    p2, s2 = mid(s1, agg1, deg3, W_neigh2, W_self2, b2r)

    agg2 = _sc_agg(p2, src_p, dst_p).reshape(NC, N_PAD, D)

    head = pl.pallas_call(
        _head_kernel,
        grid=(N // _BR,),
        in_specs=[_row_spec(D), _part_spec(D), _part_spec(D),
                  _full_spec(D, N_CLS), _full_spec(1, N_CLS)],
        out_specs=_row_spec(N_CLS),
        out_shape=jax.ShapeDtypeStruct((N, N_CLS), jnp.float32),
    )
    return head(s2, agg2, deg3, W_fc, bfr)


# speedup vs baseline: 2.8044x; 1.1614x over previous
import jax
import jax.numpy as jnp
from jax import lax
from jax.experimental import pallas as pl
from jax.experimental.pallas import tpu as pltpu
from jax.experimental.pallas import tpu_sc as plsc

N = 10000
D = 128
N_CLS = 64
E = 320000

NC = 2
NS = 16
NW = NC * NS
L = 16

CHUNK = 128
NCH = 80
EP = NW * CHUNK * NCH
XCH = 2
RPW = 640
N_PAD = NS * RPW

_MESH = plsc.VectorSubcoreMesh(core_axis_name="c", subcore_axis_name="s")


def _sc_agg_body(p_hbm, srcc, dstc, agg_out,
                 idx_s, idx_d, rows, accum,
                 sis0, sis1, sid0, sid1, sg0, sg1):
    c = lax.axis_index("c")
    s = lax.axis_index("s")
    w = c * NS + s
    base = w * NCH
    zeros16 = jnp.zeros((L,), jnp.float32)
    zbuf = rows.at[0]

    @pl.loop(0, CHUNK)
    def _(i):
        for j in range(D // L):
            zbuf[i, pl.ds(j * L, L)] = zeros16

    @pl.loop(0, RPW // CHUNK)
    def _(k):
        pltpu.sync_copy(zbuf, accum.at[pl.ds(s * RPW + k * CHUNK, CHUNK)])

    plsc.subcore_barrier()

    pltpu.sync_copy(srcc.at[base], idx_s.at[0])
    pltpu.sync_copy(dstc.at[base], idx_d.at[0])
    pltpu.async_copy(p_hbm.at[idx_s.at[0]], rows.at[0], sg0)
    pltpu.async_copy(srcc.at[base + 1], idx_s.at[1], sis1)
    pltpu.async_copy(dstc.at[base + 1], idx_d.at[1], sid1)

    @pl.loop(0, NCH, step=2)
    def _(g):
        pltpu.make_async_copy(srcc.at[base + g + 1], idx_s.at[1], sis1).wait()
        pltpu.async_copy(p_hbm.at[idx_s.at[1]], rows.at[1], sg1)
        pltpu.make_async_copy(p_hbm.at[idx_s.at[0]], rows.at[0], sg0).wait()
        pltpu.make_async_copy(dstc.at[base + g + 1], idx_d.at[1], sid1).wait()
        pltpu.sync_copy(rows.at[0], accum.at[idx_d.at[0]], add=True)
        pltpu.async_copy(srcc.at[base + g + 2], idx_s.at[0], sis0)
        pltpu.async_copy(dstc.at[base + g + 2], idx_d.at[0], sid0)

        pltpu.make_async_copy(srcc.at[base + g + 2], idx_s.at[0], sis0).wait()
        pltpu.async_copy(p_hbm.at[idx_s.at[0]], rows.at[0], sg0)
        pltpu.make_async_copy(p_hbm.at[idx_s.at[1]], rows.at[1], sg1).wait()
        pltpu.make_async_copy(dstc.at[base + g + 2], idx_d.at[0], sid0).wait()
        pltpu.sync_copy(rows.at[1], accum.at[idx_d.at[1]], add=True)
        pltpu.async_copy(srcc.at[base + g + 3], idx_s.at[1], sis1)
        pltpu.async_copy(dstc.at[base + g + 3], idx_d.at[1], sid1)

    pltpu.make_async_copy(p_hbm.at[idx_s.at[0]], rows.at[0], sg0).wait()
    pltpu.make_async_copy(srcc.at[base + NCH + 1], idx_s.at[1], sis1).wait()
    pltpu.make_async_copy(dstc.at[base + NCH + 1], idx_d.at[1], sid1).wait()

    plsc.subcore_barrier()

    @pl.loop(0, RPW // CHUNK)
    def _(k):
        r0 = s * RPW + k * CHUNK
        pltpu.sync_copy(accum.at[pl.ds(r0, CHUNK)], rows.at[0])
        pltpu.sync_copy(rows.at[0], agg_out.at[pl.ds(c * N_PAD + r0, CHUNK)])


_sc_agg = pl.kernel(
    _sc_agg_body,
    out_type=jax.ShapeDtypeStruct((NC * N_PAD, D), jnp.float32),
    mesh=_MESH,
    scratch_types=[
        pltpu.VMEM((2, CHUNK), jnp.int32),
        pltpu.VMEM((2, CHUNK), jnp.int32),
        pltpu.VMEM((2, CHUNK, D), jnp.float32),
        pltpu.VMEM_SHARED((N_PAD, D), jnp.float32),
        pltpu.SemaphoreType.DMA,
        pltpu.SemaphoreType.DMA,
        pltpu.SemaphoreType.DMA,
        pltpu.SemaphoreType.DMA,
        pltpu.SemaphoreType.DMA,
        pltpu.SemaphoreType.DMA,
    ],
)

EB = EP // 10
SUB = 2048
NHI = N_PAD // D


def _proj_kernel(x_ref, wn_ref, ws_ref, b_ref, d_ref, p_ref, s_ref, deg_ref):
    x = x_ref[...]
    p_ref[...] = jnp.dot(x, wn_ref[...], preferred_element_type=jnp.float32)
    s_ref[...] = (
        jnp.dot(x, ws_ref[...], preferred_element_type=jnp.float32) + b_ref[...]
    )

    @pl.when(pl.program_id(0) == 0)
    def _():
        deg_ref[...] = jnp.zeros((NHI, D), jnp.float32)

    acc = jnp.zeros((NHI, D), jnp.float32)
    for b in range(EB // SUB):
        db = d_ref[0, 0, pl.ds(b * SUB, SUB)]
        hi = jnp.right_shift(db, 7)
        lo = jnp.bitwise_and(db, 127)
        oh_hi = (lax.broadcasted_iota(jnp.int32, (NHI, SUB), 0)
                 == hi[None, :]).astype(jnp.bfloat16)
        oh_lo = (lax.broadcasted_iota(jnp.int32, (SUB, D), 1)
                 == lo[:, None]).astype(jnp.bfloat16)
        acc = acc + jnp.dot(oh_hi, oh_lo, preferred_element_type=jnp.float32)
    deg_ref[...] += acc


def _mid_kernel(s_ref, a_ref, d_ref, wn_ref, ws_ref, b_ref, p_ref, s2_ref):
    deg = jnp.maximum(d_ref[0, 0], 1.0)[:, None]
    agg = (a_ref[0] + a_ref[1]) / deg
    h = jnp.maximum(s_ref[...] + agg, 0.0)
    p_ref[...] = jnp.dot(h, wn_ref[...], preferred_element_type=jnp.float32)
    s2_ref[...] = (
        jnp.dot(h, ws_ref[...], preferred_element_type=jnp.float32) + b_ref[...]
    )


def _head_kernel(s_ref, a_ref, d_ref, wfc_ref, b_ref, o_ref):
    deg = jnp.maximum(d_ref[0, 0], 1.0)[:, None]
    agg = (a_ref[0] + a_ref[1]) / deg
    h = jnp.maximum(s_ref[...] + agg, 0.0)
    o_ref[...] = (
        jnp.dot(h, wfc_ref[...], preferred_element_type=jnp.float32) + b_ref[...]
    )


_BR = 1024
_GRID = -(-N // _BR)


def _row_spec(d):
    return pl.BlockSpec((_BR, d), lambda i: (i, 0))


def _part_spec(d):
    return pl.BlockSpec((2, _BR, d), lambda i: (0, i, 0))


_DEG_SPEC = pl.BlockSpec((1, 1, _BR), lambda i: (i, 0, 0))


def _full_spec(a, b):
    return pl.BlockSpec((a, b), lambda i: (0, 0))


def kernel(x, edge_index, W_self1, W_neigh1, b1, W_self2, W_neigh2, b2, W_fc, b_fc):
    src = edge_index[0].astype(jnp.int32)
    dst = edge_index[1].astype(jnp.int32)
    pad = EP - E
    xtra = XCH * CHUNK
    pad_src = (jnp.arange(pad, dtype=jnp.int32) * 2003) % N
    pad_dst = N + (jnp.arange(pad, dtype=jnp.int32) % (N_PAD - N))
    src_p = jnp.concatenate(
        [src, pad_src, jnp.zeros((xtra,), jnp.int32)]).reshape(
            NW * NCH + XCH, CHUNK)
    dst_p = jnp.concatenate(
        [dst, pad_dst, jnp.zeros((xtra,), jnp.int32)]).reshape(
            NW * NCH + XCH, CHUNK)
    b1r = b1.reshape(1, D)
    b2r = b2.reshape(1, D)
    bfr = b_fc.reshape(1, N_CLS)

    dst_e = jnp.concatenate([dst, pad_dst]).reshape(10, 1, EB)
    proj = pl.pallas_call(
        _proj_kernel,
        grid=(_GRID,),
        in_specs=[_row_spec(D), _full_spec(D, D), _full_spec(D, D),
                  _full_spec(1, D), pl.BlockSpec((1, 1, EB), lambda i: (i, 0, 0))],
        out_specs=[_row_spec(D), _row_spec(D),
                   pl.BlockSpec((NHI, D), lambda i: (0, 0))],
        out_shape=[jax.ShapeDtypeStruct((N, D), jnp.float32),
                   jax.ShapeDtypeStruct((N, D), jnp.float32),
                   jax.ShapeDtypeStruct((NHI, D), jnp.float32)],
    )
    p1, s1, deg80 = proj(x, W_neigh1, W_self1, b1r, dst_e)

    deg2 = deg80.reshape(N_PAD // _BR, 1, _BR)
    agg1 = _sc_agg(p1, src_p, dst_p).reshape(NC, N_PAD, D)

    mid = pl.pallas_call(
        _mid_kernel,
        grid=(_GRID,),
        in_specs=[_row_spec(D), _part_spec(D), _DEG_SPEC,
                  _full_spec(D, D), _full_spec(D, D), _full_spec(1, D)],
        out_specs=[_row_spec(D), _row_spec(D)],
        out_shape=[jax.ShapeDtypeStruct((N, D), jnp.float32)] * 2,
    )
    p2, s2 = mid(s1, agg1, deg2, W_neigh2, W_self2, b2r)

    agg2 = _sc_agg(p2, src_p, dst_p).reshape(NC, N_PAD, D)

    head = pl.pallas_call(
        _head_kernel,
        grid=(_GRID,),
        in_specs=[_row_spec(D), _part_spec(D), _DEG_SPEC,
                  _full_spec(D, N_CLS), _full_spec(1, N_CLS)],
        out_specs=_row_spec(N_CLS),
        out_shape=jax.ShapeDtypeStruct((N, N_CLS), jnp.float32),
    )
    return head(s2, agg2, deg2, W_fc, bfr)


# speedup vs baseline: 2.8276x; 1.0083x over previous
import jax
import jax.numpy as jnp
from jax import lax
from jax.experimental import pallas as pl
from jax.experimental.pallas import tpu as pltpu
from jax.experimental.pallas import tpu_sc as plsc

N = 10000
D = 128
N_CLS = 64
E = 320000

NC = 2
NS = 16
NW = NC * NS
L = 16

CHUNK = 128
NCH = 80
EP = NW * CHUNK * NCH
XCH = 2
RPW = 640
N_PAD = NS * RPW

_MESH = plsc.VectorSubcoreMesh(core_axis_name="c", subcore_axis_name="s")


def _sc_agg_body(p_hbm, srcc, dstc, agg_out,
                 idx_s, idx_d, rows, accum,
                 sis0, sis1, sid0, sid1, sg0, sg1):
    c = lax.axis_index("c")
    s = lax.axis_index("s")
    w = c * NS + s
    base = w * NCH
    zeros16 = jnp.zeros((L,), jnp.float32)
    zbuf = rows.at[0]

    @pl.loop(0, CHUNK)
    def _(i):
        for j in range(D // L):
            zbuf[i, pl.ds(j * L, L)] = zeros16

    for k in range(RPW // CHUNK):
        pltpu.async_copy(zbuf, accum.at[pl.ds(s * RPW + k * CHUNK, CHUNK)], sg1)
    for k in range(RPW // CHUNK):
        pltpu.make_async_copy(zbuf, accum.at[pl.ds(s * RPW + k * CHUNK, CHUNK)],
                              sg1).wait()

    plsc.subcore_barrier()

    pltpu.sync_copy(srcc.at[base], idx_s.at[0])
    pltpu.sync_copy(dstc.at[base], idx_d.at[0])
    pltpu.async_copy(p_hbm.at[idx_s.at[0]], rows.at[0], sg0)
    pltpu.async_copy(srcc.at[base + 1], idx_s.at[1], sis1)
    pltpu.async_copy(dstc.at[base + 1], idx_d.at[1], sid1)

    @pl.loop(0, NCH, step=2)
    def _(g):
        pltpu.make_async_copy(srcc.at[base + g + 1], idx_s.at[1], sis1).wait()
        pltpu.async_copy(p_hbm.at[idx_s.at[1]], rows.at[1], sg1)
        pltpu.make_async_copy(p_hbm.at[idx_s.at[0]], rows.at[0], sg0).wait()
        pltpu.make_async_copy(dstc.at[base + g + 1], idx_d.at[1], sid1).wait()
        pltpu.sync_copy(rows.at[0], accum.at[idx_d.at[0]], add=True)
        pltpu.async_copy(srcc.at[base + g + 2], idx_s.at[0], sis0)
        pltpu.async_copy(dstc.at[base + g + 2], idx_d.at[0], sid0)

        pltpu.make_async_copy(srcc.at[base + g + 2], idx_s.at[0], sis0).wait()
        pltpu.async_copy(p_hbm.at[idx_s.at[0]], rows.at[0], sg0)
        pltpu.make_async_copy(p_hbm.at[idx_s.at[1]], rows.at[1], sg1).wait()
        pltpu.make_async_copy(dstc.at[base + g + 2], idx_d.at[0], sid0).wait()
        pltpu.sync_copy(rows.at[1], accum.at[idx_d.at[1]], add=True)
        pltpu.async_copy(srcc.at[base + g + 3], idx_s.at[1], sis1)
        pltpu.async_copy(dstc.at[base + g + 3], idx_d.at[1], sid1)

    pltpu.make_async_copy(p_hbm.at[idx_s.at[0]], rows.at[0], sg0).wait()
    pltpu.make_async_copy(srcc.at[base + NCH + 1], idx_s.at[1], sis1).wait()
    pltpu.make_async_copy(dstc.at[base + NCH + 1], idx_d.at[1], sid1).wait()

    plsc.subcore_barrier()

    r0 = s * RPW
    o0 = c * N_PAD + s * RPW
    pltpu.async_copy(accum.at[pl.ds(r0, CHUNK)], rows.at[0], sg0)
    for k in range(RPW // CHUNK):
        b = k % 2
        nxt = 1 - b
        if k + 1 < RPW // CHUNK:
            pltpu.async_copy(accum.at[pl.ds(r0 + (k + 1) * CHUNK, CHUNK)],
                             rows.at[nxt], sg1 if nxt else sg0)
        pltpu.make_async_copy(accum.at[pl.ds(r0 + k * CHUNK, CHUNK)],
                              rows.at[b], sg1 if b else sg0).wait()
        pltpu.sync_copy(rows.at[b], agg_out.at[pl.ds(o0 + k * CHUNK, CHUNK)])


_sc_agg = pl.kernel(
    _sc_agg_body,
    out_type=jax.ShapeDtypeStruct((NC * N_PAD, D), jnp.float32),
    mesh=_MESH,
    scratch_types=[
        pltpu.VMEM((2, CHUNK), jnp.int32),
        pltpu.VMEM((2, CHUNK), jnp.int32),
        pltpu.VMEM((2, CHUNK, D), jnp.float32),
        pltpu.VMEM_SHARED((N_PAD, D), jnp.float32),
        pltpu.SemaphoreType.DMA,
        pltpu.SemaphoreType.DMA,
        pltpu.SemaphoreType.DMA,
        pltpu.SemaphoreType.DMA,
        pltpu.SemaphoreType.DMA,
        pltpu.SemaphoreType.DMA,
    ],
)

EB = EP // 10
SUB = 2048
NHI = N_PAD // D


def _proj_kernel(x_ref, wn_ref, ws_ref, b_ref, d_ref, p_ref, s_ref, deg_ref):
    x = x_ref[...]
    p_ref[...] = jnp.dot(x, wn_ref[...], preferred_element_type=jnp.float32)
    s_ref[...] = (
        jnp.dot(x, ws_ref[...], preferred_element_type=jnp.float32) + b_ref[...]
    )

    @pl.when(pl.program_id(0) == 0)
    def _():
        deg_ref[...] = jnp.zeros((NHI, D), jnp.float32)

    acc = jnp.zeros((NHI, D), jnp.float32)
    for b in range(EB // SUB):
        db = d_ref[0, 0, pl.ds(b * SUB, SUB)]
        hi = jnp.right_shift(db, 7)
        lo = jnp.bitwise_and(db, 127)
        oh_hi = (lax.broadcasted_iota(jnp.int32, (NHI, SUB), 0)
                 == hi[None, :]).astype(jnp.bfloat16)
        oh_lo = (lax.broadcasted_iota(jnp.int32, (SUB, D), 1)
                 == lo[:, None]).astype(jnp.bfloat16)
        acc = acc + jnp.dot(oh_hi, oh_lo, preferred_element_type=jnp.float32)
    deg_ref[...] += acc


def _mid_kernel(s_ref, a_ref, d_ref, wn_ref, ws_ref, b_ref, p_ref, s2_ref):
    deg = jnp.maximum(d_ref[0, 0], 1.0)[:, None]
    agg = (a_ref[0] + a_ref[1]) / deg
    h = jnp.maximum(s_ref[...] + agg, 0.0)
    p_ref[...] = jnp.dot(h, wn_ref[...], preferred_element_type=jnp.float32)
    s2_ref[...] = (
        jnp.dot(h, ws_ref[...], preferred_element_type=jnp.float32) + b_ref[...]
    )


def _head_kernel(s_ref, a_ref, d_ref, wfc_ref, b_ref, o_ref):
    deg = jnp.maximum(d_ref[0, 0], 1.0)[:, None]
    agg = (a_ref[0] + a_ref[1]) / deg
    h = jnp.maximum(s_ref[...] + agg, 0.0)
    o_ref[...] = (
        jnp.dot(h, wfc_ref[...], preferred_element_type=jnp.float32) + b_ref[...]
    )


_BR = 1024
_GRID = -(-N // _BR)


def _row_spec(d):
    return pl.BlockSpec((_BR, d), lambda i: (i, 0))


def _part_spec(d):
    return pl.BlockSpec((2, _BR, d), lambda i: (0, i, 0))


_DEG_SPEC = pl.BlockSpec((1, 1, _BR), lambda i: (i, 0, 0))


def _full_spec(a, b):
    return pl.BlockSpec((a, b), lambda i: (0, 0))


def kernel(x, edge_index, W_self1, W_neigh1, b1, W_self2, W_neigh2, b2, W_fc, b_fc):
    src = edge_index[0].astype(jnp.int32)
    dst = edge_index[1].astype(jnp.int32)
    pad = EP - E
    xtra = XCH * CHUNK
    pad_src = (jnp.arange(pad, dtype=jnp.int32) * 2003) % N
    pad_dst = N + (jnp.arange(pad, dtype=jnp.int32) % (N_PAD - N))
    src_p = jnp.concatenate(
        [src, pad_src, jnp.zeros((xtra,), jnp.int32)]).reshape(
            NW * NCH + XCH, CHUNK)
    dst_p = jnp.concatenate(
        [dst, pad_dst, jnp.zeros((xtra,), jnp.int32)]).reshape(
            NW * NCH + XCH, CHUNK)
    b1r = b1.reshape(1, D)
    b2r = b2.reshape(1, D)
    bfr = b_fc.reshape(1, N_CLS)

    dst_e = jnp.concatenate([dst, pad_dst]).reshape(10, 1, EB)
    proj = pl.pallas_call(
        _proj_kernel,
        grid=(_GRID,),
        in_specs=[_row_spec(D), _full_spec(D, D), _full_spec(D, D),
                  _full_spec(1, D), pl.BlockSpec((1, 1, EB), lambda i: (i, 0, 0))],
        out_specs=[_row_spec(D), _row_spec(D),
                   pl.BlockSpec((NHI, D), lambda i: (0, 0))],
        out_shape=[jax.ShapeDtypeStruct((N, D), jnp.float32),
                   jax.ShapeDtypeStruct((N, D), jnp.float32),
                   jax.ShapeDtypeStruct((NHI, D), jnp.float32)],
    )
    p1, s1, deg80 = proj(x, W_neigh1, W_self1, b1r, dst_e)

    deg2 = deg80.reshape(N_PAD // _BR, 1, _BR)
    agg1 = _sc_agg(p1, src_p, dst_p).reshape(NC, N_PAD, D)

    mid = pl.pallas_call(
        _mid_kernel,
        grid=(_GRID,),
        in_specs=[_row_spec(D), _part_spec(D), _DEG_SPEC,
                  _full_spec(D, D), _full_spec(D, D), _full_spec(1, D)],
        out_specs=[_row_spec(D), _row_spec(D)],
        out_shape=[jax.ShapeDtypeStruct((N, D), jnp.float32)] * 2,
    )
    p2, s2 = mid(s1, agg1, deg2, W_neigh2, W_self2, b2r)

    agg2 = _sc_agg(p2, src_p, dst_p).reshape(NC, N_PAD, D)

    head = pl.pallas_call(
        _head_kernel,
        grid=(_GRID,),
        in_specs=[_row_spec(D), _part_spec(D), _DEG_SPEC,
                  _full_spec(D, N_CLS), _full_spec(1, N_CLS)],
        out_specs=_row_spec(N_CLS),
        out_shape=jax.ShapeDtypeStruct((N, N_CLS), jnp.float32),
    )
    return head(s2, agg2, deg2, W_fc, bfr)


# speedup vs baseline: 3.0694x; 1.0855x over previous
import jax
import jax.numpy as jnp
import numpy as np
from jax import lax
from jax.experimental import pallas as pl
from jax.experimental.pallas import tpu as pltpu
from jax.experimental.pallas import tpu_sc as plsc

N = 10000
D = 128
N_CLS = 64
E = 320000

NC = 2
NS = 16
NW = NC * NS
L = 16

CHUNK = 128
NCH = 80
EP = NW * CHUNK * NCH
XCH = 2
RPW = 640
N_PAD = NS * RPW

_MESH = plsc.VectorSubcoreMesh(core_axis_name="c", subcore_axis_name="s")

_NPAD_E = EP - E
_PAD_SRC = (np.arange(_NPAD_E, dtype=np.int32) * 2003) % N
_PAD_DST = (N + np.arange(_NPAD_E, dtype=np.int32) % (N_PAD - N)).astype(np.int32)


def _sc_agg_body(p_hbm, srcc, dstc, agg_out,
                 idx_s, idx_d, rows, accum,
                 sis0, sis1, sid0, sid1, sg0, sg1):
    c = lax.axis_index("c")
    s = lax.axis_index("s")
    w = c * NS + s
    base = w * NCH
    zeros16 = jnp.zeros((L,), jnp.float32)
    zbuf = rows.at[0]

    @pl.loop(0, CHUNK)
    def _(i):
        for j in range(D // L):
            zbuf[i, pl.ds(j * L, L)] = zeros16

    for k in range(RPW // CHUNK):
        pltpu.async_copy(zbuf, accum.at[pl.ds(s * RPW + k * CHUNK, CHUNK)], sg1)
    for k in range(RPW // CHUNK):
        pltpu.make_async_copy(zbuf, accum.at[pl.ds(s * RPW + k * CHUNK, CHUNK)],
                              sg1).wait()

    plsc.subcore_barrier()

    pltpu.sync_copy(srcc.at[base], idx_s.at[0])
    pltpu.sync_copy(dstc.at[base], idx_d.at[0])
    pltpu.async_copy(p_hbm.at[idx_s.at[0]], rows.at[0], sg0)
    pltpu.async_copy(srcc.at[base + 1], idx_s.at[1], sis1)
    pltpu.async_copy(dstc.at[base + 1], idx_d.at[1], sid1)

    @pl.loop(0, NCH, step=2)
    def _(g):
        pltpu.make_async_copy(srcc.at[base + g + 1], idx_s.at[1], sis1).wait()
        pltpu.async_copy(p_hbm.at[idx_s.at[1]], rows.at[1], sg1)
        pltpu.make_async_copy(p_hbm.at[idx_s.at[0]], rows.at[0], sg0).wait()
        pltpu.make_async_copy(dstc.at[base + g + 1], idx_d.at[1], sid1).wait()
        pltpu.sync_copy(rows.at[0], accum.at[idx_d.at[0]], add=True)
        pltpu.async_copy(srcc.at[base + g + 2], idx_s.at[0], sis0)
        pltpu.async_copy(dstc.at[base + g + 2], idx_d.at[0], sid0)

        pltpu.make_async_copy(srcc.at[base + g + 2], idx_s.at[0], sis0).wait()
        pltpu.async_copy(p_hbm.at[idx_s.at[0]], rows.at[0], sg0)
        pltpu.make_async_copy(p_hbm.at[idx_s.at[1]], rows.at[1], sg1).wait()
        pltpu.make_async_copy(dstc.at[base + g + 2], idx_d.at[0], sid0).wait()
        pltpu.sync_copy(rows.at[1], accum.at[idx_d.at[1]], add=True)
        pltpu.async_copy(srcc.at[base + g + 3], idx_s.at[1], sis1)
        pltpu.async_copy(dstc.at[base + g + 3], idx_d.at[1], sid1)

    pltpu.make_async_copy(p_hbm.at[idx_s.at[0]], rows.at[0], sg0).wait()
    pltpu.make_async_copy(srcc.at[base + NCH + 1], idx_s.at[1], sis1).wait()
    pltpu.make_async_copy(dstc.at[base + NCH + 1], idx_d.at[1], sid1).wait()

    plsc.subcore_barrier()

    r0 = s * RPW
    o0 = c * N_PAD + s * RPW
    pltpu.async_copy(accum.at[pl.ds(r0, CHUNK)], rows.at[0], sg0)
    for k in range(RPW // CHUNK):
        b = k % 2
        nxt = 1 - b
        if k + 1 < RPW // CHUNK:
            pltpu.async_copy(accum.at[pl.ds(r0 + (k + 1) * CHUNK, CHUNK)],
                             rows.at[nxt], sg1 if nxt else sg0)
        pltpu.make_async_copy(accum.at[pl.ds(r0 + k * CHUNK, CHUNK)],
                              rows.at[b], sg1 if b else sg0).wait()
        pltpu.sync_copy(rows.at[b], agg_out.at[pl.ds(o0 + k * CHUNK, CHUNK)])


_sc_agg = pl.kernel(
    _sc_agg_body,
    out_type=jax.ShapeDtypeStruct((NC * N_PAD, D), jnp.float32),
    mesh=_MESH,
    scratch_types=[
        pltpu.VMEM((2, CHUNK), jnp.int32),
        pltpu.VMEM((2, CHUNK), jnp.int32),
        pltpu.VMEM((2, CHUNK, D), jnp.float32),
        pltpu.VMEM_SHARED((N_PAD, D), jnp.float32),
        pltpu.SemaphoreType.DMA,
        pltpu.SemaphoreType.DMA,
        pltpu.SemaphoreType.DMA,
        pltpu.SemaphoreType.DMA,
        pltpu.SemaphoreType.DMA,
        pltpu.SemaphoreType.DMA,
    ],
)

EB = EP // 10
SUB = 2048
NHI = N_PAD // D


def _proj_kernel(x_ref, wn_ref, ws_ref, b_ref, d_ref, p_ref, s_ref, deg_ref):
    x = x_ref[...]
    p_ref[...] = jnp.dot(x, wn_ref[...], preferred_element_type=jnp.float32)
    s_ref[...] = (
        jnp.dot(x, ws_ref[...], preferred_element_type=jnp.float32) + b_ref[...]
    )

    @pl.when(pl.program_id(0) == 0)
    def _():
        deg_ref[...] = jnp.zeros((D, NHI), jnp.float32)

    acc = jnp.zeros((D, NHI), jnp.float32)
    for b in range(EB // SUB):
        db = d_ref[0, 0, pl.ds(b * SUB, SUB)]
        hi = jnp.right_shift(db, 7)
        lo = jnp.bitwise_and(db, 127)
        oh_hi = (lax.broadcasted_iota(jnp.int32, (NHI, SUB), 0)
                 == hi[None, :]).astype(jnp.bfloat16)
        oh_lo_t = (lax.broadcasted_iota(jnp.int32, (D, SUB), 0)
                   == lo[None, :]).astype(jnp.bfloat16)
        acc = acc + lax.dot_general(
            oh_lo_t, oh_hi, (((1,), (1,)), ((), ())),
            preferred_element_type=jnp.float32)
    deg_ref[...] += acc


def _mid_kernel(s_ref, a_ref, d_ref, wn_ref, ws_ref, b_ref, p_ref, s2_ref):
    deg = jnp.maximum(d_ref[0, 0], 1.0)[:, None]
    agg = (a_ref[0] + a_ref[1]) / deg
    h = jnp.maximum(s_ref[...] + agg, 0.0)
    p_ref[...] = jnp.dot(h, wn_ref[...], preferred_element_type=jnp.float32)
    s2_ref[...] = (
        jnp.dot(h, ws_ref[...], preferred_element_type=jnp.float32) + b_ref[...]
    )


def _head_kernel(s_ref, a_ref, d_ref, wfc_ref, b_ref, o_ref):
    deg = jnp.maximum(d_ref[0, 0], 1.0)[:, None]
    agg = (a_ref[0] + a_ref[1]) / deg
    h = jnp.maximum(s_ref[...] + agg, 0.0)
    o_ref[...] = (
        jnp.dot(h, wfc_ref[...], preferred_element_type=jnp.float32) + b_ref[...]
    )


_BR = 1024
_GRID = -(-N // _BR)


def _row_spec(d):
    return pl.BlockSpec((_BR, d), lambda i: (i, 0))


def _part_spec(d):
    return pl.BlockSpec((2, _BR, d), lambda i: (0, i, 0))


_DEG_SPEC = pl.BlockSpec((1, 1, _BR), lambda i: (i, 0, 0))


def _full_spec(a, b):
    return pl.BlockSpec((a, b), lambda i: (0, 0))


def kernel(x, edge_index, W_self1, W_neigh1, b1, W_self2, W_neigh2, b2, W_fc, b_fc):
    src = edge_index[0].astype(jnp.int32)
    dst = edge_index[1].astype(jnp.int32)
    pad = EP - E
    xtra = XCH * CHUNK
    pad_src = jnp.asarray(_PAD_SRC)
    pad_dst = jnp.asarray(_PAD_DST)
    src_p = jnp.concatenate(
        [src, pad_src, jnp.zeros((xtra,), jnp.int32)]).reshape(
            NW * NCH + XCH, CHUNK)
    dst_p = jnp.concatenate(
        [dst, pad_dst, jnp.zeros((xtra,), jnp.int32)]).reshape(
            NW * NCH + XCH, CHUNK)
    b1r = b1.reshape(1, D)
    b2r = b2.reshape(1, D)
    bfr = b_fc.reshape(1, N_CLS)

    dst_e = jnp.concatenate([dst, pad_dst]).reshape(10, 1, EB)
    proj = pl.pallas_call(
        _proj_kernel,
        grid=(_GRID,),
        in_specs=[_row_spec(D), _full_spec(D, D), _full_spec(D, D),
                  _full_spec(1, D), pl.BlockSpec((1, 1, EB), lambda i: (i, 0, 0))],
        out_specs=[_row_spec(D), _row_spec(D),
                   pl.BlockSpec((D, NHI), lambda i: (0, 0))],
        out_shape=[jax.ShapeDtypeStruct((N, D), jnp.float32),
                   jax.ShapeDtypeStruct((N, D), jnp.float32),
                   jax.ShapeDtypeStruct((D, NHI), jnp.float32)],
    )
    p1, s1, deg80 = proj(x, W_neigh1, W_self1, b1r, dst_e)

    deg2 = deg80.T.reshape(N_PAD // _BR, 1, _BR)
    agg1 = _sc_agg(p1, src_p, dst_p).reshape(NC, N_PAD, D)

    mid = pl.pallas_call(
        _mid_kernel,
        grid=(_GRID,),
        in_specs=[_row_spec(D), _part_spec(D), _DEG_SPEC,
                  _full_spec(D, D), _full_spec(D, D), _full_spec(1, D)],
        out_specs=[_row_spec(D), _row_spec(D)],
        out_shape=[jax.ShapeDtypeStruct((N, D), jnp.float32)] * 2,
    )
    p2, s2 = mid(s1, agg1, deg2, W_neigh2, W_self2, b2r)

    agg2 = _sc_agg(p2, src_p, dst_p).reshape(NC, N_PAD, D)

    head = pl.pallas_call(
        _head_kernel,
        grid=(_GRID,),
        in_specs=[_row_spec(D), _part_spec(D), _DEG_SPEC,
                  _full_spec(D, N_CLS), _full_spec(1, N_CLS)],
        out_specs=_row_spec(N_CLS),
        out_shape=jax.ShapeDtypeStruct((N, N_CLS), jnp.float32),
    )
    return head(s2, agg2, deg2, W_fc, bfr)
